# baseline (speedup 1.0000x reference)
import functools

import jax
import jax.numpy as jnp
from jax.experimental import pallas as pl
from jax.experimental.pallas import tpu as pltpu

BM = 1024
BN = 1024
BK = 1024


def _mm_kernel(x_ref, w_ref, b_ref, o_ref, acc_ref, *, nk):
    k = pl.program_id(2)

    @pl.when(k == 0)
    def _init():
        acc_ref[...] = jnp.zeros_like(acc_ref)

    acc_ref[...] += jax.lax.dot_general(
        x_ref[...], w_ref[...],
        (((1,), (1,)), ((), ())),
        preferred_element_type=jnp.float32)

    @pl.when(k == nk - 1)
    def _done():
        o_ref[...] = acc_ref[...] + b_ref[...]


def kernel(input, W, bias):
    B, S, K = input.shape
    N = W.shape[0]
    M = B * S
    x = input.reshape(M, K).astype(jnp.bfloat16)
    w = W.astype(jnp.bfloat16)
    b2 = bias.reshape(1, N)
    nm, nn, nk = M // BM, N // BN, K // BK
    out = pl.pallas_call(
        functools.partial(_mm_kernel, nk=nk),
        grid=(nn, nm, nk),
        in_specs=[
            pl.BlockSpec((BM, BK), lambda n, m, k: (m, k)),
            pl.BlockSpec((BN, BK), lambda n, m, k: (n, k)),
            pl.BlockSpec((1, BN), lambda n, m, k: (0, n)),
        ],
        out_specs=pl.BlockSpec((BM, BN), lambda n, m, k: (m, n)),
        out_shape=jax.ShapeDtypeStruct((M, N), jnp.float32),
        scratch_shapes=[pltpu.VMEM((BM, BN), jnp.float32)],
        compiler_params=pltpu.CompilerParams(
            dimension_semantics=("parallel", "parallel", "arbitrary"),
        ),
    )(x, w, b2)
    return out.reshape(B, S, N)


# speedup vs baseline: 1.4862x; 1.4862x over previous
import jax
import jax.numpy as jnp
from jax.experimental import pallas as pl
from jax.experimental.pallas import tpu as pltpu

BM = 256
CHR = 256
N_FEAT = 4096


def _body(x_ref, w_hbm, b_ref, o_ref, w_bf, st0, st1, s0, s1):
    m = pl.program_id(0)

    @pl.when(m == 0)
    def _load_w():
        stages = (st0, st1)
        sems = (s0, s1)
        nch = N_FEAT // CHR
        pltpu.make_async_copy(
            w_hbm.at[pl.ds(0, CHR), :], stages[0], sems[0]).start()
        for c in range(nch):
            if c + 1 < nch:
                pltpu.make_async_copy(
                    w_hbm.at[pl.ds((c + 1) * CHR, CHR), :],
                    stages[(c + 1) % 2], sems[(c + 1) % 2]).start()
            pltpu.make_async_copy(
                w_hbm.at[pl.ds(c * CHR, CHR), :],
                stages[c % 2], sems[c % 2]).wait()
            w_bf[pl.ds(c * CHR, CHR), :] = stages[c % 2][...].astype(jnp.bfloat16)

    xb = x_ref[...].astype(jnp.bfloat16)
    o_ref[...] = jax.lax.dot_general(
        xb, w_bf[...],
        (((1,), (1,)), ((), ())),
        preferred_element_type=jnp.float32) + b_ref[...]


def kernel(input, W, bias):
    B, S, K = input.shape
    N = W.shape[0]
    M = B * S
    x = input.reshape(M, K)
    b2 = bias.reshape(1, N)
    nm = M // BM
    out = pl.pallas_call(
        _body,
        grid=(nm,),
        in_specs=[
            pl.BlockSpec((BM, K), lambda m: (m, 0)),
            pl.BlockSpec(memory_space=pltpu.MemorySpace.HBM),
            pl.BlockSpec((1, N), lambda m: (0, 0)),
        ],
        out_specs=pl.BlockSpec((BM, N), lambda m: (m, 0)),
        out_shape=jax.ShapeDtypeStruct((M, N), jnp.float32),
        scratch_shapes=[
            pltpu.VMEM((N, K), jnp.bfloat16),
            pltpu.VMEM((CHR, K), jnp.float32),
            pltpu.VMEM((CHR, K), jnp.float32),
            pltpu.SemaphoreType.DMA,
            pltpu.SemaphoreType.DMA,
        ],
        compiler_params=pltpu.CompilerParams(
            dimension_semantics=("arbitrary",),
            vmem_limit_bytes=100 * 1024 * 1024,
        ),
    )(x, W, b2)
    return out.reshape(B, S, N)
